# baseline (speedup 1.0000x reference)
import jax
import jax.numpy as jnp
from jax.experimental import pallas as pl
from jax.experimental.pallas import tpu as pltpu
from jax.experimental.pallas import tpu_sc as plsc

_BM = 16
_LANES = 16


def _sc_body(x_vmem, row_vmem, o_vmem):
    @pl.loop(0, _BM)
    def _(r):
        @pl.loop(0, x_vmem.shape[1], step=_LANES)
        def _(c):
            slc = (pl.ds(r, 1), pl.ds(c, _LANES))
            o_vmem.at[*slc][...] = x_vmem.at[*slc][...] + row_vmem.at[*slc][...]


def kernel(x, row_embed):
    B, L, D = x.shape
    table = row_embed[:L]
    x2 = x.reshape(B * L, D)
    n_row_blocks = L // _BM
    mesh = plsc.VectorSubcoreMesh(core_axis_name="c", subcore_axis_name="s")

    @pl.kernel(
        out_type=jax.ShapeDtypeStruct((B * L, D), x.dtype),
        mesh=mesh,
        scratch_types=[],
    )
    def sc_kernel(x_hbm, row_hbm, o_hbm):
        pltpu.emit_pipeline(
            _sc_body,
            grid=(B * L // _BM,),
            in_specs=[
                pl.BlockSpec((_BM, D), index_map=lambda i: (i, 0)),
                pl.BlockSpec((_BM, D), index_map=lambda i: (i % n_row_blocks, 0)),
            ],
            out_specs=[pl.BlockSpec((_BM, D), index_map=lambda i: (i, 0))],
            core_axis_name=("c", "s"),
            dimension_semantics=(pltpu.PARALLEL,),
        )(x_hbm, row_hbm, o_hbm)

    return sc_kernel(x2, table).reshape(B, L, D)
